# baseline (speedup 1.0000x reference)
import functools

import jax
import jax.numpy as jnp
from jax import lax
from jax.experimental import pallas as pl
from jax.experimental.pallas import tpu as pltpu
from jax.experimental.pallas import tpu_sc as plsc

_LANES = 16


def _bf16_hi(u):
    return (u + jnp.uint32(0x7FFF) + ((u >> 16) & jnp.uint32(1))) & jnp.uint32(0xFFFF0000)


def _pack_tables_body(g_ref, b_ref, o_ref):
    ug = lax.bitcast_convert_type(g_ref[...], jnp.uint32)
    ub = lax.bitcast_convert_type(b_ref[...], jnp.uint32)
    o_ref[...] = _bf16_hi(ug) | (_bf16_hi(ub) >> 16)


def _pack_tables(g, b):
    return pl.pallas_call(
        _pack_tables_body,
        out_shape=jax.ShapeDtypeStruct(g.shape, jnp.uint32),
    )(g, b)


@functools.lru_cache(maxsize=None)
def _build(B, F, V, C):
    info = plsc.get_sparse_core_info()
    NC, NS = info.num_cores, info.num_subcores
    NW = NC * NS
    b_per_w = B // NW
    n_chunks = b_per_w // C
    vpr = F // _LANES
    _HI = jnp.uint32(0xFFFF0000)

    mesh = plsc.VectorSubcoreMesh(core_axis_name="c", subcore_axis_name="s")

    @functools.partial(
        pl.kernel,
        mesh=mesh,
        out_type=jax.ShapeDtypeStruct((B, F), jnp.float32),
        scratch_types=[
            pltpu.VMEM((b_per_w,), jnp.int32),
            pltpu.VMEM((C, F), jnp.uint32),
            pltpu.VMEM((C, F), jnp.float32),
            pltpu.VMEM((C, F), jnp.float32),
            pltpu.VMEM((C, F), jnp.uint32),
            pltpu.VMEM((C, F), jnp.float32),
            pltpu.VMEM((C, F), jnp.float32),
            pltpu.SemaphoreType.DMA,
            pltpu.SemaphoreType.DMA,
            pltpu.SemaphoreType.DMA,
            pltpu.SemaphoreType.DMA,
        ],
    )
    def film(x_hbm, idx_hbm, t_hbm, out_hbm,
             idx_v, p0, x0, o0, p1, x1, o1, si0, si1, so0, so1):
        P, X, O = (p0, p1), (x0, x1), (o0, o1)
        SI, SO = (si0, si1), (so0, so1)
        wid = lax.axis_index("s") * NC + lax.axis_index("c")
        base = wid * b_per_w
        pltpu.sync_copy(idx_hbm.at[pl.ds(base, b_per_w)], idx_v)

        def start_in(c, s):
            o = pl.multiple_of(c * C, 8)
            pltpu.async_copy(t_hbm.at[idx_v.at[pl.ds(o, C)]], P[s], SI[s])
            ox = pl.multiple_of(base + c * C, 8)
            pltpu.async_copy(x_hbm.at[pl.ds(ox, C)], X[s], SI[s])

        def wait_in(s):
            pltpu.make_async_copy(t_hbm.at[idx_v.at[pl.ds(0, C)]], P[s], SI[s]).wait()
            pltpu.make_async_copy(x_hbm.at[pl.ds(base, C)], X[s], SI[s]).wait()

        def start_out(c, s):
            pltpu.async_copy(O[s], out_hbm.at[pl.ds(base + c * C, C)], SO[s])

        def wait_out(s):
            pltpu.make_async_copy(O[s], out_hbm.at[pl.ds(base, C)], SO[s]).wait()

        start_in(0, 0)
        start_in(1, 1)

        @pl.loop(0, n_chunks // 2)
        def pair(p):
            for s in (0, 1):
                c = 2 * p + s
                wait_in(s)

                @pl.when(p > 0)
                def _():
                    wait_out(s)

                pv, xv, ov = P[s], X[s], O[s]

                @plsc.parallel_loop(0, C, unroll=8)
                def row(r):
                    for j in range(vpr):
                        sl = pl.ds(j * _LANES, _LANES)
                        t = pv[r, sl]
                        g = lax.bitcast_convert_type(t & _HI, jnp.float32)
                        b = lax.bitcast_convert_type(t << 16, jnp.float32)
                        ov[r, sl] = g * xv[r, sl] + b

                start_out(c, s)

                @pl.when(c + 2 < n_chunks)
                def _():
                    start_in(c + 2, s)

        wait_out(0)
        wait_out(1)

    return film


@jax.jit
def kernel(x, cell_lines, gammas, betas):
    B, F = x.shape
    V = gammas.shape[0]
    idx = cell_lines.astype(jnp.int32)
    packed_t = _pack_tables(gammas, betas)
    return _build(B, F, V, 64)(x, idx, packed_t)


# speedup vs baseline: 1.0013x; 1.0013x over previous
import functools

import jax
import jax.numpy as jnp
from jax import lax
from jax.experimental import pallas as pl
from jax.experimental.pallas import tpu as pltpu
from jax.experimental.pallas import tpu_sc as plsc

_LANES = 16


def _bf16_hi(u):
    return (u + jnp.uint32(0x7FFF) + ((u >> 16) & jnp.uint32(1))) & jnp.uint32(0xFFFF0000)


def _pack_tables_body(g_ref, b_ref, o_ref):
    ug = lax.bitcast_convert_type(g_ref[...], jnp.uint32)
    ub = lax.bitcast_convert_type(b_ref[...], jnp.uint32)
    o_ref[...] = _bf16_hi(ug) | (_bf16_hi(ub) >> 16)


def _pack_tables(g, b):
    return pl.pallas_call(
        _pack_tables_body,
        out_shape=jax.ShapeDtypeStruct(g.shape, jnp.uint32),
    )(g, b)


@functools.lru_cache(maxsize=None)
def _build(B, F, V, C):
    info = plsc.get_sparse_core_info()
    NC, NS = info.num_cores, info.num_subcores
    NW = NC * NS
    b_per_w = B // NW
    n_chunks = b_per_w // C
    vpr = F // _LANES
    _HI = jnp.uint32(0xFFFF0000)

    mesh = plsc.VectorSubcoreMesh(core_axis_name="c", subcore_axis_name="s")

    @functools.partial(
        pl.kernel,
        mesh=mesh,
        out_type=jax.ShapeDtypeStruct((B, F), jnp.float32),
        scratch_types=[
            pltpu.VMEM((b_per_w,), jnp.int32),
            pltpu.VMEM((C, F), jnp.uint32),
            pltpu.VMEM((C, F), jnp.float32),
            pltpu.VMEM((C, F), jnp.float32),
            pltpu.VMEM((C, F), jnp.uint32),
            pltpu.VMEM((C, F), jnp.float32),
            pltpu.VMEM((C, F), jnp.float32),
            pltpu.VMEM((C, F), jnp.uint32),
            pltpu.VMEM((C, F), jnp.float32),
            pltpu.VMEM((C, F), jnp.float32),
            pltpu.VMEM((C, F), jnp.uint32),
            pltpu.VMEM((C, F), jnp.float32),
            pltpu.VMEM((C, F), jnp.float32),
            pltpu.SemaphoreType.DMA,
            pltpu.SemaphoreType.DMA,
            pltpu.SemaphoreType.DMA,
            pltpu.SemaphoreType.DMA,
            pltpu.SemaphoreType.DMA,
            pltpu.SemaphoreType.DMA,
            pltpu.SemaphoreType.DMA,
            pltpu.SemaphoreType.DMA,
        ],
    )
    def film(x_hbm, idx_hbm, t_hbm, out_hbm,
             idx_v, p0, x0, o0, p1, x1, o1, p2, x2, o2, p3, x3, o3,
             si0, si1, si2, si3, so0, so1, so2, so3):
        P, X, O = (p0, p1, p2, p3), (x0, x1, x2, x3), (o0, o1, o2, o3)
        SI, SO = (si0, si1, si2, si3), (so0, so1, so2, so3)
        wid = lax.axis_index("s") * NC + lax.axis_index("c")
        base = wid * b_per_w
        pltpu.sync_copy(idx_hbm.at[pl.ds(base, b_per_w)], idx_v)

        def start_in(c, s):
            o = pl.multiple_of(c * C, 8)
            pltpu.async_copy(t_hbm.at[idx_v.at[pl.ds(o, C)]], P[s], SI[s])
            ox = pl.multiple_of(base + c * C, 8)
            pltpu.async_copy(x_hbm.at[pl.ds(ox, C)], X[s], SI[s])

        def wait_in(s):
            pltpu.make_async_copy(t_hbm.at[idx_v.at[pl.ds(0, C)]], P[s], SI[s]).wait()
            pltpu.make_async_copy(x_hbm.at[pl.ds(base, C)], X[s], SI[s]).wait()

        def start_out(c, s):
            pltpu.async_copy(O[s], out_hbm.at[pl.ds(base + c * C, C)], SO[s])

        def wait_out(s):
            pltpu.make_async_copy(O[s], out_hbm.at[pl.ds(base, C)], SO[s]).wait()

        for s in (0, 1, 2, 3):
            start_in(s, s)

        @pl.loop(0, n_chunks // 4)
        def quad(p):
            for s in (0, 1, 2, 3):
                c = 4 * p + s
                wait_in(s)

                @pl.when(p > 0)
                def _():
                    wait_out(s)

                pv, xv, ov = P[s], X[s], O[s]

                @plsc.parallel_loop(0, C, unroll=4)
                def row(r):
                    for j in range(vpr):
                        sl = pl.ds(j * _LANES, _LANES)
                        t = pv[r, sl]
                        g = lax.bitcast_convert_type(t & _HI, jnp.float32)
                        b = lax.bitcast_convert_type(t << 16, jnp.float32)
                        ov[r, sl] = g * xv[r, sl] + b

                start_out(c, s)

                @pl.when(c + 4 < n_chunks)
                def _():
                    start_in(c + 4, s)

        for s in (0, 1, 2, 3):
            wait_out(s)

    return film


@jax.jit
def kernel(x, cell_lines, gammas, betas):
    B, F = x.shape
    V = gammas.shape[0]
    idx = cell_lines.astype(jnp.int32)
    packed_t = _pack_tables(gammas, betas)
    return _build(B, F, V, 64)(x, idx, packed_t)


# speedup vs baseline: 1.0179x; 1.0166x over previous
import functools

import jax
import jax.numpy as jnp
from jax import lax
from jax.experimental import pallas as pl
from jax.experimental.pallas import tpu as pltpu
from jax.experimental.pallas import tpu_sc as plsc

_LANES = 16


def _bf16_hi(u):
    return (u + jnp.uint32(0x7FFF) + ((u >> 16) & jnp.uint32(1))) & jnp.uint32(0xFFFF0000)


def _pack_tables_body(g_ref, b_ref, o_ref):
    ug = lax.bitcast_convert_type(g_ref[...], jnp.uint32)
    ub = lax.bitcast_convert_type(b_ref[...], jnp.uint32)
    o_ref[...] = _bf16_hi(ug) | (_bf16_hi(ub) >> 16)


def _pack_tables(g, b):
    return pl.pallas_call(
        _pack_tables_body,
        out_shape=jax.ShapeDtypeStruct(g.shape, jnp.uint32),
    )(g, b)


@functools.lru_cache(maxsize=None)
def _build(B, F, V, C):
    info = plsc.get_sparse_core_info()
    NC, NS = info.num_cores, info.num_subcores
    NW = NC * NS
    b_per_w = B // NW
    n_chunks = b_per_w // C
    vpr = F // _LANES
    _HI = jnp.uint32(0xFFFF0000)

    mesh = plsc.VectorSubcoreMesh(core_axis_name="c", subcore_axis_name="s")

    @functools.partial(
        pl.kernel,
        mesh=mesh,
        out_type=jax.ShapeDtypeStruct((B, F), jnp.float32),
        scratch_types=[
            pltpu.VMEM((b_per_w,), jnp.int32),
            pltpu.VMEM((C, F), jnp.uint32),
            pltpu.VMEM((C, F), jnp.float32),
            pltpu.VMEM((C, F), jnp.float32),
            pltpu.VMEM((C, F), jnp.uint32),
            pltpu.VMEM((C, F), jnp.float32),
            pltpu.VMEM((C, F), jnp.float32),
            pltpu.SemaphoreType.DMA,
            pltpu.SemaphoreType.DMA,
            pltpu.SemaphoreType.DMA,
            pltpu.SemaphoreType.DMA,
        ],
    )
    def film(x_hbm, idx_hbm, t_hbm, out_hbm,
             idx_v, p0, x0, o0, p1, x1, o1, si0, si1, so0, so1):
        P, X, O = (p0, p1), (x0, x1), (o0, o1)
        SI, SO = (si0, si1), (so0, so1)
        wid = lax.axis_index("s") * NC + lax.axis_index("c")
        base = wid * b_per_w
        pltpu.sync_copy(idx_hbm.at[pl.ds(base, b_per_w)], idx_v)

        def start_in(c, s):
            o = pl.multiple_of(c * C, 8)
            pltpu.async_copy(t_hbm.at[idx_v.at[pl.ds(o, C)]], P[s], SI[s])
            ox = pl.multiple_of(base + c * C, 8)
            pltpu.async_copy(x_hbm.at[pl.ds(ox, C)], X[s], SI[s])

        def wait_in(s):
            pltpu.make_async_copy(t_hbm.at[idx_v.at[pl.ds(0, C)]], P[s], SI[s]).wait()
            pltpu.make_async_copy(x_hbm.at[pl.ds(base, C)], X[s], SI[s]).wait()

        def start_out(c, s):
            pltpu.async_copy(O[s], out_hbm.at[pl.ds(base + c * C, C)], SO[s])

        def wait_out(s):
            pltpu.make_async_copy(O[s], out_hbm.at[pl.ds(base, C)], SO[s]).wait()

        start_in(0, 0)
        start_in(1, 1)

        @pl.loop(0, n_chunks // 2)
        def pair(p):
            for s in (0, 1):
                c = 2 * p + s
                wait_in(s)

                @pl.when(p > 0)
                def _():
                    wait_out(s)

                pv, xv, ov = P[s], X[s], O[s]

                @plsc.parallel_loop(0, C, unroll=4)
                def row(r):
                    for j in range(vpr):
                        sl = pl.ds(j * _LANES, _LANES)
                        t = pv[r, sl]
                        g = lax.bitcast_convert_type(t & _HI, jnp.float32)
                        b = lax.bitcast_convert_type(t << 16, jnp.float32)
                        ov[r, sl] = g * xv[r, sl] + b

                start_out(c, s)

                @pl.when(c + 2 < n_chunks)
                def _():
                    start_in(c + 2, s)

        wait_out(0)
        wait_out(1)

    return film


@jax.jit
def kernel(x, cell_lines, gammas, betas):
    B, F = x.shape
    V = gammas.shape[0]
    idx = cell_lines.astype(jnp.int32)
    packed_t = _pack_tables(gammas, betas)
    return _build(B, F, V, 64)(x, idx, packed_t)
